# Initial kernel scaffold; baseline (speedup 1.0000x reference)
import jax
import jax.numpy as jnp
from jax.experimental import pallas as pl


def kernel(words, masks, pos, ner, deprel, head, subj_pos, obj_pos, subj_type, obj_type, trees, max_depth, emb_w, pos_w, ner_w, W_iou, U_iou, b_iou, W_f, U_f, b_f, mlp_w, mlp_b, cls_w, cls_b):
    raise NotImplementedError("write your pallas kernel here")


# speedup vs baseline: 4.0556x; 4.0556x over previous
import functools

import jax
import jax.numpy as jnp
from jax import lax
from jax.experimental import pallas as pl
from jax.experimental.pallas import tpu as pltpu
from jax.experimental.pallas import tpu_sc as plsc

B = 32
T = 64
MEM = 128
R = T + 2
NN = B * T
NTBL = NN + 8
INDIM = 360
DEPTH = 4

NC = 2
NS = 16
NW = NC * NS
NPW = NN // NW
LANES = 16
DB = MEM // LANES


def _sc_level_body(tbl_hbm, idx_hbm, nxf_hbm, out_hbm,
                   idx_v, nxf_v, rows0, rows1, out_v, sem0, sem1):
    wid = lax.axis_index("s") * NC + lax.axis_index("c")
    base = wid * NPW
    pltpu.sync_copy(idx_hbm.at[pl.ds(base, NPW)], idx_v)
    pltpu.sync_copy(nxf_hbm.at[pl.ds(base, NPW)], nxf_v)

    def accumulate(n, rows):
        nxf = [nxf_v[n, pl.ds(db * LANES, LANES)] for db in range(DB)]

        def child(s, carry):
            acc = list(carry)
            for db in range(DB):
                hrow = rows[s, pl.ds(db * LANES, LANES)]
                nfh = rows[s, pl.ds(MEM + db * LANES, LANES)]
                crow = rows[s, pl.ds(2 * MEM + db * LANES, LANES)]
                e = jnp.exp(nxf[db] + nfh)
                acc[db] = acc[db] + hrow
                acc[DB + db] = acc[DB + db] + crow / (1.0 + e)
            return tuple(acc)

        zero = jnp.zeros((LANES,), jnp.float32)
        acc = lax.fori_loop(0, T, child, (zero,) * (2 * DB))
        for db in range(DB):
            out_v[n, pl.ds(db * LANES, LANES)] = acc[db]
            out_v[n, pl.ds(MEM + db * LANES, LANES)] = acc[DB + db]

    pltpu.async_copy(tbl_hbm.at[idx_v.at[0]], rows0, sem0)

    def outer(g, _):
        n0 = 2 * g
        n1 = n0 + 1
        pltpu.make_async_copy(tbl_hbm.at[idx_v.at[n0]], rows0, sem0).wait()
        pltpu.async_copy(tbl_hbm.at[idx_v.at[n1]], rows1, sem1)
        accumulate(n0, rows0)
        pltpu.make_async_copy(tbl_hbm.at[idx_v.at[n1]], rows1, sem1).wait()

        @pl.when(n1 + 1 < NPW)
        def _():
            pltpu.async_copy(tbl_hbm.at[idx_v.at[n1 + 1]], rows0, sem0)

        accumulate(n1, rows1)
        return 0

    lax.fori_loop(0, NPW // 2, outer, 0)
    pltpu.sync_copy(out_v, out_hbm.at[pl.ds(base, NPW)])


@functools.cache
def _make_sc_level():
    return pl.kernel(
        _sc_level_body,
        out_type=jax.ShapeDtypeStruct((NN, 2 * MEM), jnp.float32),
        mesh=plsc.VectorSubcoreMesh(core_axis_name="c", subcore_axis_name="s",
                                    num_cores=NC, num_subcores=NS),
        scratch_types=[
            pltpu.VMEM((NPW, T), jnp.int32),
            pltpu.VMEM((NPW, MEM), jnp.float32),
            pltpu.VMEM((T, 3 * MEM), jnp.float32),
            pltpu.VMEM((T, 3 * MEM), jnp.float32),
            pltpu.VMEM((NPW, 2 * MEM), jnp.float32),
            pltpu.SemaphoreType.DMA,
            pltpu.SemaphoreType.DMA,
        ],
    )


def _sc_level(tbl, idx2, nxf):
    return _make_sc_level()(tbl, idx2, nxf)


def _gates(iou):
    i = jax.nn.sigmoid(iou[:, :MEM])
    o = jax.nn.sigmoid(iou[:, MEM:2 * MEM])
    u = jnp.tanh(iou[:, 2 * MEM:])
    return i, o, u


def _write_tbl(tbl_ref, h, nfh, c):
    tbl_ref[pl.ds(0, NN), :] = jnp.concatenate([h, nfh, c], axis=1)
    tbl_ref[pl.ds(NN, NTBL - NN), :] = jnp.zeros((NTBL - NN, 3 * MEM),
                                                 jnp.float32)


def _prologue_body(x_ref, wiou_ref, biou_ref, wf_ref, uf_ref, bf_ref,
                   trees_ref, xw_ref, nxf_ref, tbl_ref, idx2_ref):
    x = x_ref[...]
    xw = jnp.dot(x, wiou_ref[...], preferred_element_type=jnp.float32)
    xw = xw + biou_ref[...]
    xw_ref[...] = xw
    nxf_ref[...] = -jnp.dot(x, wf_ref[...], preferred_element_type=jnp.float32)
    i, o, u = _gates(xw)
    c1 = i * u
    h1 = o * jnp.tanh(c1)
    nfh = -(jnp.dot(h1, uf_ref[...], preferred_element_type=jnp.float32)
            + bf_ref[...])
    _write_tbl(tbl_ref, h1, nfh, c1)
    t = trees_ref[...]
    bidx = t // R
    r = t - bidx * R
    idx2_ref[...] = jnp.where(r >= 2, bidx * T + (r - 2), NN)


_prologue = pl.pallas_call(
    _prologue_body,
    out_shape=(
        jax.ShapeDtypeStruct((NN, 3 * MEM), jnp.float32),
        jax.ShapeDtypeStruct((NN, MEM), jnp.float32),
        jax.ShapeDtypeStruct((NTBL, 3 * MEM), jnp.float32),
        jax.ShapeDtypeStruct((NN, T), jnp.int32),
    ),
)


def _cell_mid_body(xw_ref, sums_ref, uiou_ref, uf_ref, bf_ref, tbl_ref):
    sums = sums_ref[...]
    iou = xw_ref[...] + jnp.dot(sums[:, :MEM], uiou_ref[...],
                                preferred_element_type=jnp.float32)
    i, o, u = _gates(iou)
    c = i * u + sums[:, MEM:]
    h = o * jnp.tanh(c)
    nfh = -(jnp.dot(h, uf_ref[...], preferred_element_type=jnp.float32)
            + bf_ref[...])
    _write_tbl(tbl_ref, h, nfh, c)


_cell_mid = pl.pallas_call(
    _cell_mid_body,
    out_shape=jax.ShapeDtypeStruct((NTBL, 3 * MEM), jnp.float32),
)


def _cell_last_body(xw_ref, sums_ref, uiou_ref, h_ref):
    sums = sums_ref[...]
    iou = xw_ref[...] + jnp.dot(sums[:, :MEM], uiou_ref[...],
                                preferred_element_type=jnp.float32)
    i, o, u = _gates(iou)
    c = i * u + sums[:, MEM:]
    h_ref[...] = o * jnp.tanh(c)


_cell_last = pl.pallas_call(
    _cell_last_body,
    out_shape=jax.ShapeDtypeStruct((NN, MEM), jnp.float32),
)


def _head_body(h_ref, masks_ref, subj_ref, obj_ref,
               mlpw_ref, mlpb_ref, clsw_ref, clsb_ref, out_ref):
    h3 = h_ref[...].reshape(B, T, MEM)
    pm = masks_ref[...] != 0

    def pool(mk):
        mk3 = lax.broadcast_in_dim(mk.astype(jnp.float32), (B, T, MEM), (0, 1))
        return jnp.max(h3 * (1.0 - mk3) + (-1e12) * mk3, axis=1)

    outs = jnp.concatenate(
        [pool(pm), pool(pm | (subj_ref[...] != 0)),
         pool(pm | (obj_ref[...] != 0))], axis=1)
    hid = jnp.dot(outs, mlpw_ref[...], preferred_element_type=jnp.float32)
    hid = jnp.maximum(hid + mlpb_ref[...], 0.0)
    out_ref[...] = (jnp.dot(hid, clsw_ref[...],
                            preferred_element_type=jnp.float32)
                    + clsb_ref[...])


_head = pl.pallas_call(
    _head_body,
    out_shape=jax.ShapeDtypeStruct((B, MEM), jnp.float32),
)


def kernel(words, masks, pos, ner, deprel, head, subj_pos, obj_pos,
           subj_type, obj_type, trees, max_depth,
           emb_w, pos_w, ner_w, W_iou, U_iou, b_iou, W_f, U_f, b_f,
           mlp_w, mlp_b, cls_w, cls_b):
    x = jnp.concatenate(
        [jnp.take(emb_w, words.reshape(-1), axis=0),
         jnp.take(pos_w, pos.reshape(-1), axis=0),
         jnp.take(ner_w, ner.reshape(-1), axis=0)], axis=1)

    cls_w_p = jnp.zeros((MEM, MEM), jnp.float32).at[:, :42].set(cls_w)
    cls_b_p = jnp.zeros((1, MEM), jnp.float32).at[0, :42].set(cls_b)

    xw, nxf, tbl, idx2 = _prologue(
        x, W_iou, b_iou.reshape(1, -1), W_f, U_f, b_f.reshape(1, -1),
        trees.reshape(NN, T))

    for level in range(1, DEPTH):
        sums = _sc_level(tbl, idx2, nxf)
        if level < DEPTH - 1:
            tbl = _cell_mid(xw, sums, U_iou, U_f, b_f.reshape(1, -1))
        else:
            h_final = _cell_last(xw, sums, U_iou)

    logits = _head(h_final, masks, subj_pos, obj_pos,
                   mlp_w, mlp_b.reshape(1, -1), cls_w_p, cls_b_p)
    return logits[:, :42]
